# baseline (speedup 1.0000x reference)
import functools

import jax
import jax.numpy as jnp
from jax import lax
from jax.experimental import pallas as pl
from jax.experimental.pallas import tpu as pltpu
from jax.experimental.pallas import tpu_sc as plsc

D_MODEL = 4096
N_EXP = 4
TOK_TILE = 1024
SC_WORKERS = 32
SC_LANES = 16


def _moe_body(x_ref, wt5_ref, b_ref, g_ref, rv_ref, o_ref):
    xb = x_ref[...]
    r = jnp.dot(xb, wt5_ref[...], preferred_element_type=jnp.float32)
    logits = r[:, :N_EXP]
    s = r[:, N_EXP]
    z = (logits + b_ref[...]) + g_ref[...]
    idx = jnp.argmax(z, axis=-1)

    nzm = (xb != 0.0).astype(jnp.bfloat16)
    cmpm = (xb != jnp.roll(xb, 1, axis=1)).astype(jnp.bfloat16)
    nz = jnp.dot(nzm, rv_ref[:, :1],
                 preferred_element_type=jnp.float32)[:, 0]
    wdp = jnp.dot(cmpm, rv_ref[:, 1:],
                  preferred_element_type=jnp.float32)
    wd = wdp[:, 0] + wdp[:, 1]

    o_ref[0, :] = s
    o_ref[1, :] = (s / (s + 1e-6)) / jnp.float32(D_MODEL)
    o_ref[2, :] = nz
    o_ref[3, :] = wd / jnp.float32(D_MODEL)
    o_ref[4, :] = idx.astype(jnp.float32)
    o_ref[5, :] = s
    o_ref[6, :] = s
    o_ref[7, :] = s


def _tc_stage(x2, wt5, b2, g, rv):
    n_tok, D = x2.shape
    grid = (n_tok // TOK_TILE,)
    return pl.pallas_call(
        _moe_body,
        grid=grid,
        in_specs=[
            pl.BlockSpec((TOK_TILE, D), lambda i: (i, 0)),
            pl.BlockSpec((D, N_EXP + 1), lambda i: (0, 0)),
            pl.BlockSpec((1, N_EXP), lambda i: (0, 0)),
            pl.BlockSpec((TOK_TILE, N_EXP), lambda i: (i, 0)),
            pl.BlockSpec((D, 3), lambda i: (0, 0)),
        ],
        out_specs=pl.BlockSpec((8, TOK_TILE), lambda i: (0, i)),
        out_shape=jax.ShapeDtypeStruct((8, n_tok), jnp.float32),
        compiler_params=pltpu.CompilerParams(
            dimension_semantics=("parallel",)),
    )(x2, wt5, b2, g, rv)


def _run(x2, W, b, g, rv):
    n_tok, D = x2.shape
    wt5 = jnp.concatenate([W.T, jnp.ones((D, 1), jnp.float32)], axis=1)
    b2 = b.reshape(1, N_EXP)
    half = n_tok // 2
    ev1 = _tc_stage(x2[:half], wt5, b2, g[:half], rv)
    ev2 = _tc_stage(x2[half:], wt5, b2, g[half:], rv)
    return jnp.concatenate([_route_sc(ev1), _route_sc(ev2)])


def _route_sc(ev):
    n_tok = ev.shape[1]
    per = n_tok // SC_WORKERS
    mesh = plsc.VectorSubcoreMesh(core_axis_name="c", subcore_axis_name="s")

    @functools.partial(
        pl.kernel, mesh=mesh,
        out_type=jax.ShapeDtypeStruct((n_tok,), jnp.float32),
        scratch_types=[
            pltpu.VMEM((5 * per,), jnp.float32),
            pltpu.VMEM((per,), jnp.float32),
        ],
    )
    def k(ev_hbm, out_hbm, ev_v, out_v):
        wid = lax.axis_index("s") * 2 + lax.axis_index("c")
        base = wid * per
        for r in range(5):
            pltpu.sync_copy(ev_hbm.at[r, pl.ds(base, per)],
                            ev_v.at[pl.ds(r * per, per)])

        def body(j, carry):
            off = j * SC_LANES
            sl = pl.ds(off, SC_LANES)
            idxf = ev_v[pl.ds(4 * per + off, SC_LANES)]
            out = ev_v[pl.ds(off, SC_LANES)]
            for r in range(1, N_EXP):
                er = ev_v[pl.ds(r * per + off, SC_LANES)]
                out = jnp.where(idxf == jnp.float32(r), er, out)
            out_v[sl] = out
            return carry

        lax.fori_loop(0, per // SC_LANES, body, 0)
        pltpu.sync_copy(out_v, out_hbm.at[pl.ds(base, per)])

    return k(ev)


_run_jit = jax.jit(_run)
_consts = {}


def _get_consts(B, S, D):
    key = (B, S, D)
    if key not in _consts:
        g = jax.random.gumbel(
            jax.random.key(42), (B, S, N_EXP), dtype=jnp.float32
        ).reshape(B * S, N_EXP)
        i = jnp.arange(D, dtype=jnp.float32)
        w = jnp.where(i == 0, 0.0, jnp.float32(D) - i)
        w_hi = w.astype(jnp.bfloat16).astype(jnp.float32)
        w_lo = w - w_hi
        rv = jnp.stack([jnp.ones((D,), jnp.float32), w_hi, w_lo], axis=1)
        _consts[key] = (g, rv.astype(jnp.bfloat16))
    return _consts[key]


def kernel(x, W, b):
    B, S, D = x.shape
    g, rv = _get_consts(B, S, D)
    out = _run_jit(x.reshape(B * S, D), W, b, g, rv)
    return out.reshape(B, S, 1)


# speedup vs baseline: 1.7109x; 1.7109x over previous
import functools

import jax
import jax.numpy as jnp
from jax import lax
from jax.experimental import pallas as pl
from jax.experimental.pallas import tpu as pltpu
from jax.experimental.pallas import tpu_sc as plsc

D_MODEL = 4096
N_EXP = 4
TOK_TILE = 1024
SC_WORKERS = 32
SC_LANES = 16


def _moe_body(x_ref, wt5_ref, b_ref, g_ref, rv_ref, o_ref):
    xb = x_ref[...]
    r = jnp.dot(xb, wt5_ref[...], preferred_element_type=jnp.float32)
    logits = r[:, :N_EXP]
    s = r[:, N_EXP]
    z = (logits + b_ref[...]) + g_ref[...]
    idx = jnp.argmax(z, axis=-1)

    nzm = (xb != 0.0).astype(jnp.bfloat16)
    cmpm = (xb != jnp.roll(xb, 1, axis=1)).astype(jnp.bfloat16)
    nz = jnp.dot(nzm, rv_ref[:, :1],
                 preferred_element_type=jnp.float32)[:, 0]
    wdp = jnp.dot(cmpm, rv_ref[:, 1:],
                  preferred_element_type=jnp.float32)
    wd = wdp[:, 0] + wdp[:, 1]

    o_ref[0, :] = s
    o_ref[1, :] = (s / (s + 1e-6)) / jnp.float32(D_MODEL)
    o_ref[2, :] = nz
    o_ref[3, :] = wd / jnp.float32(D_MODEL)
    o_ref[4, :] = idx.astype(jnp.float32)
    o_ref[5, :] = s
    o_ref[6, :] = s
    o_ref[7, :] = s


def _tc_stage(x2, wt5, b2, g, rv, half_idx):
    n_tok, D = x2.shape
    half_tiles = n_tok // TOK_TILE // 2
    base = half_idx * half_tiles
    return pl.pallas_call(
        _moe_body,
        grid=(half_tiles,),
        in_specs=[
            pl.BlockSpec((TOK_TILE, D), lambda i: (base + i, 0)),
            pl.BlockSpec((D, N_EXP + 1), lambda i: (0, 0)),
            pl.BlockSpec((1, N_EXP), lambda i: (0, 0)),
            pl.BlockSpec((TOK_TILE, N_EXP), lambda i: (base + i, 0)),
            pl.BlockSpec((D, 3), lambda i: (0, 0)),
        ],
        out_specs=pl.BlockSpec((8, TOK_TILE), lambda i: (0, i)),
        out_shape=jax.ShapeDtypeStruct((8, n_tok // 2), jnp.float32),
        compiler_params=pltpu.CompilerParams(
            dimension_semantics=("parallel",)),
    )(x2, wt5, b2, g, rv)


def _run(x2, W, b, g, rv):
    n_tok, D = x2.shape
    wt5 = jnp.concatenate([W.T, jnp.ones((D, 1), jnp.float32)], axis=1)
    b2 = b.reshape(1, N_EXP)
    ev1 = _tc_stage(x2, wt5, b2, g, rv, 0)
    ev2 = _tc_stage(x2, wt5, b2, g, rv, 1)
    return jnp.concatenate([_route_sc(ev1), _route_sc(ev2)])


def _route_sc(ev):
    n_tok = ev.shape[1]
    per = n_tok // SC_WORKERS
    mesh = plsc.VectorSubcoreMesh(core_axis_name="c", subcore_axis_name="s")

    @functools.partial(
        pl.kernel, mesh=mesh,
        out_type=jax.ShapeDtypeStruct((n_tok,), jnp.float32),
        scratch_types=[
            pltpu.VMEM((5 * per,), jnp.float32),
            pltpu.VMEM((per,), jnp.float32),
        ],
    )
    def k(ev_hbm, out_hbm, ev_v, out_v):
        wid = lax.axis_index("s") * 2 + lax.axis_index("c")
        base = wid * per
        for r in range(5):
            pltpu.sync_copy(ev_hbm.at[r, pl.ds(base, per)],
                            ev_v.at[pl.ds(r * per, per)])

        def body(j, carry):
            off = j * SC_LANES
            sl = pl.ds(off, SC_LANES)
            idxf = ev_v[pl.ds(4 * per + off, SC_LANES)]
            out = ev_v[pl.ds(off, SC_LANES)]
            for r in range(1, N_EXP):
                er = ev_v[pl.ds(r * per + off, SC_LANES)]
                out = jnp.where(idxf == jnp.float32(r), er, out)
            out_v[sl] = out
            return carry

        lax.fori_loop(0, per // SC_LANES, body, 0)
        pltpu.sync_copy(out_v, out_hbm.at[pl.ds(base, per)])

    return k(ev)


_run_jit = jax.jit(_run)
_consts = {}


def _get_consts(B, S, D):
    key = (B, S, D)
    if key not in _consts:
        g = jax.random.gumbel(
            jax.random.key(42), (B, S, N_EXP), dtype=jnp.float32
        ).reshape(B * S, N_EXP)
        i = jnp.arange(D, dtype=jnp.float32)
        w = jnp.where(i == 0, 0.0, jnp.float32(D) - i)
        w_hi = w.astype(jnp.bfloat16).astype(jnp.float32)
        w_lo = w - w_hi
        rv = jnp.stack([jnp.ones((D,), jnp.float32), w_hi, w_lo], axis=1)
        _consts[key] = (g, rv.astype(jnp.bfloat16))
    return _consts[key]


def kernel(x, W, b):
    B, S, D = x.shape
    g, rv = _get_consts(B, S, D)
    out = _run_jit(x.reshape(B * S, D), W, b, g, rv)
    return out.reshape(B, S, 1)


# speedup vs baseline: 1.8341x; 1.0720x over previous
import functools

import jax
import jax.numpy as jnp
from jax import lax
from jax.experimental import pallas as pl
from jax.experimental.pallas import tpu as pltpu
from jax.experimental.pallas import tpu_sc as plsc

D_MODEL = 4096
N_EXP = 4
TOK_TILE = 1024
SC_WORKERS = 32
SC_LANES = 16


def _moe_body(x_ref, wt5_ref, b_ref, g_ref, rv_ref, o_ref):
    xb = x_ref[...]
    r = jnp.dot(xb, wt5_ref[...], preferred_element_type=jnp.float32)
    logits = r[:, :N_EXP]
    s = r[:, N_EXP]
    z = (logits + b_ref[...]) + g_ref[...]
    idx = jnp.argmax(z, axis=-1)

    nzm = (xb != 0.0).astype(jnp.bfloat16)
    cmpm = (xb != jnp.roll(xb, 1, axis=1)).astype(jnp.bfloat16)
    nz = jnp.dot(nzm, rv_ref[:, :1],
                 preferred_element_type=jnp.float32)[:, 0]
    wdp = jnp.dot(cmpm, rv_ref[:, 1:],
                  preferred_element_type=jnp.float32)
    wd = wdp[:, 0] + wdp[:, 1]

    o_ref[0, :] = s
    o_ref[1, :] = (s / (s + 1e-6)) / jnp.float32(D_MODEL)
    o_ref[2, :] = nz
    o_ref[3, :] = wd / jnp.float32(D_MODEL)
    o_ref[4, :] = idx.astype(jnp.float32)
    o_ref[5, :] = s
    o_ref[6, :] = s
    o_ref[7, :] = s


def _run(x2, W, b, g, rv):
    n_tok, D = x2.shape
    wt5 = jnp.concatenate([W.T, jnp.ones((D, 1), jnp.float32)], axis=1)
    b2 = b.reshape(1, N_EXP)
    ev = pl.pallas_call(
        _moe_body,
        grid=(n_tok // TOK_TILE,),
        in_specs=[
            pl.BlockSpec((TOK_TILE, D), lambda i: (i, 0)),
            pl.BlockSpec((D, N_EXP + 1), lambda i: (0, 0)),
            pl.BlockSpec((1, N_EXP), lambda i: (0, 0)),
            pl.BlockSpec((TOK_TILE, N_EXP), lambda i: (i, 0)),
            pl.BlockSpec((D, 3), lambda i: (0, 0)),
        ],
        out_specs=pl.BlockSpec((8, TOK_TILE), lambda i: (0, i)),
        out_shape=jax.ShapeDtypeStruct((8, n_tok), jnp.float32),
        compiler_params=pltpu.CompilerParams(
            dimension_semantics=("parallel",)),
    )(x2, wt5, b2, g, rv)
    return _route_sc(ev)


def _route_sc(ev):
    n_tok = ev.shape[1]
    per = n_tok // SC_WORKERS
    mesh = plsc.VectorSubcoreMesh(core_axis_name="c", subcore_axis_name="s")

    @functools.partial(
        pl.kernel, mesh=mesh,
        out_type=jax.ShapeDtypeStruct((n_tok,), jnp.float32),
        scratch_types=[
            pltpu.VMEM((5 * per,), jnp.float32),
            pltpu.VMEM((per,), jnp.float32),
        ],
    )
    def k(ev_hbm, out_hbm, ev_v, out_v):
        wid = lax.axis_index("s") * 2 + lax.axis_index("c")
        base = wid * per
        for r in range(5):
            pltpu.sync_copy(ev_hbm.at[r, pl.ds(base, per)],
                            ev_v.at[pl.ds(r * per, per)])

        def body(j, carry):
            off = j * SC_LANES
            sl = pl.ds(off, SC_LANES)
            idxf = ev_v[pl.ds(4 * per + off, SC_LANES)]
            out = ev_v[pl.ds(off, SC_LANES)]
            for r in range(1, N_EXP):
                er = ev_v[pl.ds(r * per + off, SC_LANES)]
                out = jnp.where(idxf == jnp.float32(r), er, out)
            out_v[sl] = out
            return carry

        lax.fori_loop(0, per // SC_LANES, body, 0)
        pltpu.sync_copy(out_v, out_hbm.at[pl.ds(base, per)])

    return k(ev)


_run_jit = jax.jit(_run)
_consts = {}


def _get_consts(B, S, D):
    key = (B, S, D)
    if key not in _consts:
        g = jax.random.gumbel(
            jax.random.key(42), (B, S, N_EXP), dtype=jnp.float32
        ).reshape(B * S, N_EXP)
        i = jnp.arange(D, dtype=jnp.float32)
        w = jnp.where(i == 0, 0.0, jnp.float32(D) - i)
        w_hi = w.astype(jnp.bfloat16).astype(jnp.float32)
        w_lo = w - w_hi
        rv = jnp.stack([jnp.ones((D,), jnp.float32), w_hi, w_lo], axis=1)
        _consts[key] = (g, rv.astype(jnp.bfloat16))
    return _consts[key]


def kernel(x, W, b):
    B, S, D = x.shape
    g, rv = _get_consts(B, S, D)
    out = _run_jit(x.reshape(B * S, D), W, b, g, rv)
    return out.reshape(B, S, 1)


# speedup vs baseline: 1.8384x; 1.0023x over previous
import functools

import jax
import jax.numpy as jnp
from jax import lax
from jax.experimental import pallas as pl
from jax.experimental.pallas import tpu as pltpu
from jax.experimental.pallas import tpu_sc as plsc

D_MODEL = 4096
N_EXP = 4
TOK_TILE = 512
SC_WORKERS = 32
SC_LANES = 16


def _moe_body(x_ref, wt5_ref, b_ref, g_ref, rv_ref, o_ref):
    xb = x_ref[...]
    r = jnp.dot(xb, wt5_ref[...], preferred_element_type=jnp.float32)
    logits = r[:, :N_EXP]
    s = r[:, N_EXP]
    z = (logits + b_ref[...]) + g_ref[...]
    idx = jnp.argmax(z, axis=-1)

    nzm = (xb != 0.0).astype(jnp.bfloat16)
    cmpm = (xb != jnp.roll(xb, 1, axis=1)).astype(jnp.bfloat16)
    nz = jnp.dot(nzm, rv_ref[:, :1],
                 preferred_element_type=jnp.float32)[:, 0]
    wdp = jnp.dot(cmpm, rv_ref[:, 1:],
                  preferred_element_type=jnp.float32)
    wd = wdp[:, 0] + wdp[:, 1]

    o_ref[0, :] = s
    o_ref[1, :] = (s / (s + 1e-6)) / jnp.float32(D_MODEL)
    o_ref[2, :] = nz
    o_ref[3, :] = wd / jnp.float32(D_MODEL)
    o_ref[4, :] = idx.astype(jnp.float32)
    o_ref[5, :] = s
    o_ref[6, :] = s
    o_ref[7, :] = s


def _run(x2, W, b, g, rv):
    n_tok, D = x2.shape
    wt5 = jnp.concatenate([W.T, jnp.ones((D, 1), jnp.float32)], axis=1)
    b2 = b.reshape(1, N_EXP)
    ev = pl.pallas_call(
        _moe_body,
        grid=(n_tok // TOK_TILE,),
        in_specs=[
            pl.BlockSpec((TOK_TILE, D), lambda i: (i, 0)),
            pl.BlockSpec((D, N_EXP + 1), lambda i: (0, 0)),
            pl.BlockSpec((1, N_EXP), lambda i: (0, 0)),
            pl.BlockSpec((TOK_TILE, N_EXP), lambda i: (i, 0)),
            pl.BlockSpec((D, 3), lambda i: (0, 0)),
        ],
        out_specs=pl.BlockSpec((8, TOK_TILE), lambda i: (0, i)),
        out_shape=jax.ShapeDtypeStruct((8, n_tok), jnp.float32),
        compiler_params=pltpu.CompilerParams(
            dimension_semantics=("parallel",)),
    )(x2, wt5, b2, g, rv)
    return _route_sc(ev)


def _route_sc(ev):
    n_tok = ev.shape[1]
    per = n_tok // SC_WORKERS
    mesh = plsc.VectorSubcoreMesh(core_axis_name="c", subcore_axis_name="s")

    @functools.partial(
        pl.kernel, mesh=mesh,
        out_type=jax.ShapeDtypeStruct((n_tok,), jnp.float32),
        scratch_types=[
            pltpu.VMEM((5 * per,), jnp.float32),
            pltpu.VMEM((per,), jnp.float32),
        ],
    )
    def k(ev_hbm, out_hbm, ev_v, out_v):
        wid = lax.axis_index("s") * 2 + lax.axis_index("c")
        base = wid * per
        for r in range(5):
            pltpu.sync_copy(ev_hbm.at[r, pl.ds(base, per)],
                            ev_v.at[pl.ds(r * per, per)])

        def body(j, carry):
            off = j * SC_LANES
            sl = pl.ds(off, SC_LANES)
            idxf = ev_v[pl.ds(4 * per + off, SC_LANES)]
            out = ev_v[pl.ds(off, SC_LANES)]
            for r in range(1, N_EXP):
                er = ev_v[pl.ds(r * per + off, SC_LANES)]
                out = jnp.where(idxf == jnp.float32(r), er, out)
            out_v[sl] = out
            return carry

        lax.fori_loop(0, per // SC_LANES, body, 0)
        pltpu.sync_copy(out_v, out_hbm.at[pl.ds(base, per)])

    return k(ev)


_run_jit = jax.jit(_run)
_consts = {}


def _get_consts(B, S, D):
    key = (B, S, D)
    if key not in _consts:
        g = jax.random.gumbel(
            jax.random.key(42), (B, S, N_EXP), dtype=jnp.float32
        ).reshape(B * S, N_EXP)
        i = jnp.arange(D, dtype=jnp.float32)
        w = jnp.where(i == 0, 0.0, jnp.float32(D) - i)
        w_hi = w.astype(jnp.bfloat16).astype(jnp.float32)
        w_lo = w - w_hi
        rv = jnp.stack([jnp.ones((D,), jnp.float32), w_hi, w_lo], axis=1)
        _consts[key] = (g, rv.astype(jnp.bfloat16))
    return _consts[key]


def kernel(x, W, b):
    B, S, D = x.shape
    g, rv = _get_consts(B, S, D)
    out = _run_jit(x.reshape(B * S, D), W, b, g, rv)
    return out.reshape(B, S, 1)


# speedup vs baseline: 1.8665x; 1.0153x over previous
import functools

import jax
import jax.numpy as jnp
from jax import lax
from jax.experimental import pallas as pl
from jax.experimental.pallas import tpu as pltpu
from jax.experimental.pallas import tpu_sc as plsc

D_MODEL = 4096
N_EXP = 4
TOK_TILE = 512
SC_WORKERS = 32
SC_LANES = 16


def _moe_body(x_ref, wt5_ref, b_ref, g_ref, rv_ref, o_ref):
    xb = x_ref[...]
    r = jnp.dot(xb, wt5_ref[...], preferred_element_type=jnp.float32)
    logits = r[:, :N_EXP]
    s = r[:, N_EXP]
    z = (logits + b_ref[...]) + g_ref[...]
    idx = jnp.argmax(z, axis=-1)

    nzm = (xb != 0.0).astype(jnp.bfloat16)
    cmpm = (xb != jnp.roll(xb, 1, axis=1)).astype(jnp.bfloat16)
    nz = jnp.dot(nzm, rv_ref[:, :1],
                 preferred_element_type=jnp.float32)[:, 0]
    wdp = jnp.dot(cmpm, rv_ref[:, 1:],
                  preferred_element_type=jnp.float32)
    wd = wdp[:, 0] + wdp[:, 1]

    o_ref[0, :] = s
    o_ref[1, :] = (s / (s + 1e-6)) / jnp.float32(D_MODEL)
    o_ref[2, :] = nz
    o_ref[3, :] = wd / jnp.float32(D_MODEL)
    o_ref[4, :] = idx.astype(jnp.float32)
    o_ref[5, :] = s
    o_ref[6, :] = s
    o_ref[7, :] = s


def _run(x2, W, b, g, rv):
    n_tok, D = x2.shape
    wt5 = jnp.concatenate([W.T, jnp.ones((D, 1), jnp.float32)], axis=1)
    b2 = b.reshape(1, N_EXP)
    ev = pl.pallas_call(
        _moe_body,
        grid=(n_tok // TOK_TILE,),
        in_specs=[
            pl.BlockSpec((TOK_TILE, D), lambda i: (i, 0)),
            pl.BlockSpec((D, N_EXP + 1), lambda i: (0, 0)),
            pl.BlockSpec((1, N_EXP), lambda i: (0, 0)),
            pl.BlockSpec((TOK_TILE, N_EXP), lambda i: (i, 0)),
            pl.BlockSpec((D, 3), lambda i: (0, 0)),
        ],
        out_specs=pl.BlockSpec((8, TOK_TILE), lambda i: (0, i)),
        out_shape=jax.ShapeDtypeStruct((8, n_tok), jnp.float32),
        compiler_params=pltpu.CompilerParams(
            dimension_semantics=("parallel",)),
    )(x2, wt5, b2, g, rv)
    return _route_sc(ev)


def _route_sc(ev):
    n_tok = ev.shape[1]
    per = n_tok // SC_WORKERS
    mesh = plsc.VectorSubcoreMesh(core_axis_name="c", subcore_axis_name="s")

    @functools.partial(
        pl.kernel, mesh=mesh,
        out_type=jax.ShapeDtypeStruct((n_tok,), jnp.float32),
        scratch_types=[
            pltpu.VMEM((5, per), jnp.float32),
            pltpu.VMEM((per,), jnp.float32),
        ],
    )
    def k(ev_hbm, out_hbm, ev_v, out_v):
        wid = lax.axis_index("s") * 2 + lax.axis_index("c")
        base = wid * per
        pltpu.sync_copy(ev_hbm.at[pl.ds(0, 5), pl.ds(base, per)], ev_v)

        def body(j, carry):
            off = j * SC_LANES
            sl = pl.ds(off, SC_LANES)
            idxf = ev_v[4, sl]
            out = ev_v[0, sl]
            for r in range(1, N_EXP):
                out = jnp.where(idxf == jnp.float32(r), ev_v[r, sl], out)
            out_v[sl] = out
            return carry

        lax.fori_loop(0, per // SC_LANES, body, 0)
        pltpu.sync_copy(out_v, out_hbm.at[pl.ds(base, per)])

    return k(ev)


_run_jit = jax.jit(_run)
_consts = {}


def _get_consts(B, S, D):
    key = (B, S, D)
    if key not in _consts:
        g = jax.random.gumbel(
            jax.random.key(42), (B, S, N_EXP), dtype=jnp.float32
        ).reshape(B * S, N_EXP)
        i = jnp.arange(D, dtype=jnp.float32)
        w = jnp.where(i == 0, 0.0, jnp.float32(D) - i)
        w_hi = w.astype(jnp.bfloat16).astype(jnp.float32)
        w_lo = w - w_hi
        rv = jnp.stack([jnp.ones((D,), jnp.float32), w_hi, w_lo], axis=1)
        _consts[key] = (g, rv.astype(jnp.bfloat16))
    return _consts[key]


def kernel(x, W, b):
    B, S, D = x.shape
    g, rv = _get_consts(B, S, D)
    out = _run_jit(x.reshape(B * S, D), W, b, g, rv)
    return out.reshape(B, S, 1)


# speedup vs baseline: 1.8688x; 1.0012x over previous
import functools

import jax
import jax.numpy as jnp
from jax import lax
from jax.experimental import pallas as pl
from jax.experimental.pallas import tpu as pltpu
from jax.experimental.pallas import tpu_sc as plsc

D_MODEL = 4096
N_EXP = 4
TOK_TILE = 512
SC_WORKERS = 32
SC_LANES = 16


def _moe_body(x_ref, wt5_ref, b_ref, g_ref, rv_ref, o_ref):
    xb = x_ref[...]
    r = jnp.dot(xb, wt5_ref[...], preferred_element_type=jnp.float32)
    logits = r[:, :N_EXP]
    s = r[:, N_EXP]
    z = (logits + b_ref[...]) + g_ref[...]
    idx = jnp.argmax(z, axis=-1)

    nzm = (xb != 0.0).astype(jnp.bfloat16)
    cmpm = (xb != jnp.roll(xb, 1, axis=1)).astype(jnp.bfloat16)
    nz = jnp.dot(nzm, rv_ref[:, :1],
                 preferred_element_type=jnp.float32)[:, 0]
    wdp = jnp.dot(cmpm, rv_ref[:, 1:],
                  preferred_element_type=jnp.float32)
    wd = wdp[:, 0] + wdp[:, 1]

    o_ref[0, :] = s
    o_ref[1, :] = (s / (s + 1e-6)) / jnp.float32(D_MODEL)
    o_ref[2, :] = nz
    o_ref[3, :] = wd / jnp.float32(D_MODEL)
    o_ref[4, :] = idx.astype(jnp.float32)
    o_ref[5, :] = s
    o_ref[6, :] = s
    o_ref[7, :] = s


def _run(x2, W, b, g, rv):
    n_tok, D = x2.shape
    wt5 = jnp.concatenate([W.T, jnp.ones((D, 1), jnp.float32)], axis=1)
    b2 = b.reshape(1, N_EXP)
    ev = pl.pallas_call(
        _moe_body,
        grid=(n_tok // TOK_TILE,),
        in_specs=[
            pl.BlockSpec((TOK_TILE, D), lambda i: (i, 0)),
            pl.BlockSpec((D, N_EXP + 1), lambda i: (0, 0)),
            pl.BlockSpec((1, N_EXP), lambda i: (0, 0)),
            pl.BlockSpec((TOK_TILE, N_EXP), lambda i: (i, 0)),
            pl.BlockSpec((D, 3), lambda i: (0, 0)),
        ],
        out_specs=pl.BlockSpec((8, TOK_TILE), lambda i: (0, i)),
        out_shape=jax.ShapeDtypeStruct((8, n_tok), jnp.float32),
        compiler_params=pltpu.CompilerParams(
            dimension_semantics=("parallel",)),
    )(x2, wt5, b2, g, rv)
    return _route_sc(ev)


def _route_sc(ev):
    n_tok = ev.shape[1]
    per = n_tok // SC_WORKERS
    mesh = plsc.VectorSubcoreMesh(core_axis_name="c", subcore_axis_name="s")

    @functools.partial(
        pl.kernel, mesh=mesh,
        out_type=jax.ShapeDtypeStruct((n_tok,), jnp.float32),
        scratch_types=[
            pltpu.VMEM((5, per), jnp.float32),
            pltpu.VMEM((per,), jnp.float32),
        ],
    )
    def k(ev_hbm, out_hbm, ev_v, out_v):
        wid = lax.axis_index("s") * 2 + lax.axis_index("c")
        base = wid * per
        pltpu.sync_copy(ev_hbm.at[pl.ds(0, 5), pl.ds(base, per)], ev_v)

        for j in range(per // SC_LANES):
            sl = pl.ds(j * SC_LANES, SC_LANES)
            idxf = ev_v[4, sl]
            out = ev_v[0, sl]
            for r in range(1, N_EXP):
                out = jnp.where(idxf == jnp.float32(r), ev_v[r, sl], out)
            out_v[sl] = out

        pltpu.sync_copy(out_v, out_hbm.at[pl.ds(base, per)])

    return k(ev)


_run_jit = jax.jit(_run)
_consts = {}


def _get_consts(B, S, D):
    key = (B, S, D)
    if key not in _consts:
        g = jax.random.gumbel(
            jax.random.key(42), (B, S, N_EXP), dtype=jnp.float32
        ).reshape(B * S, N_EXP)
        i = jnp.arange(D, dtype=jnp.float32)
        w = jnp.where(i == 0, 0.0, jnp.float32(D) - i)
        w_hi = w.astype(jnp.bfloat16).astype(jnp.float32)
        w_lo = w - w_hi
        rv = jnp.stack([jnp.ones((D,), jnp.float32), w_hi, w_lo], axis=1)
        _consts[key] = (g, rv.astype(jnp.bfloat16))
    return _consts[key]


def kernel(x, W, b):
    B, S, D = x.shape
    g, rv = _get_consts(B, S, D)
    out = _run_jit(x.reshape(B * S, D), W, b, g, rv)
    return out.reshape(B, S, 1)
